# baseline (speedup 1.0000x reference)
import functools

import jax
import jax.numpy as jnp
from jax import lax
from jax.experimental import pallas as pl
from jax.experimental.pallas import tpu as pltpu
from jax.experimental.pallas import tpu_sc as plsc

B = 4096
H = 200
D = 64
NC = 2
NS = 16
NW = NC * NS
BPW = B // NW
Q = 5
NCHUNK = H // Q
NBUF = 2
LANES = 16
NG = D // LANES


CH = Q * BPW


def _pool_body(x_hbm, table_hbm, out_hbm, idx_v, buf_v, out_v, sem0, sem1):
    wid = lax.axis_index("s") * NC + lax.axis_index("c")
    base_b = wid * BPW
    pltpu.sync_copy(x_hbm.at[wid], idx_v)

    sems = (sem0, sem1)

    def issue(c):
        return pltpu.async_copy(
            table_hbm.at[idx_v.at[pl.ds(c * CH, CH)]],
            buf_v.at[c % NBUF], sems[c % NBUF])

    def accumulate(c, first):
        bk = buf_v.at[c % NBUF]

        def acc_step(r, _):
            for g in range(NG):
                o = jnp.zeros((LANES,), jnp.float32) if first \
                    else out_v[r, pl.ds(g * LANES, LANES)]
                for q in range(Q):
                    o = o + bk[q * BPW + r, pl.ds(g * LANES, LANES)]
                out_v[r, pl.ds(g * LANES, LANES)] = o
            return _

        lax.fori_loop(0, BPW, acc_step, 0)

    cps = [issue(0), issue(1)]
    for c in range(NCHUNK):
        cps[c].wait()
        if c + NBUF < NCHUNK:
            cps.append(issue(c + NBUF))
        accumulate(c, first=(c == 0))

    pltpu.sync_copy(out_v, out_hbm.at[pl.ds(base_b, BPW)])


def _pool(x_t, table):
    mesh = plsc.VectorSubcoreMesh(core_axis_name="c", subcore_axis_name="s",
                                  num_cores=NC, num_subcores=NS)
    return pl.kernel(
        _pool_body,
        out_type=jax.ShapeDtypeStruct((B, D), jnp.float32),
        mesh=mesh,
        scratch_types=[
            pltpu.VMEM((H * BPW,), jnp.int32),
            pltpu.VMEM((NBUF, CH, D), jnp.float32),
            pltpu.VMEM((BPW, D), jnp.float32),
            pltpu.SemaphoreType.DMA,
            pltpu.SemaphoreType.DMA,
        ],
        compiler_params=pltpu.CompilerParams(use_tc_tiling_on_sc=False),
    )(x_t, table)


def _mlp_body(x_ref, w1_ref, b1_ref, w2_ref, b2_ref, o_ref):
    h = lax.dot_general(x_ref[...], w1_ref[...], (((1,), (1,)), ((), ())),
                        preferred_element_type=jnp.float32)
    h = jnp.maximum(h + b1_ref[...], 0.0)
    o = lax.dot_general(h, w2_ref[...], (((1,), (1,)), ((), ())),
                        preferred_element_type=jnp.float32)
    o_ref[...] = o + b2_ref[...]


def _mlp(pooled, W1, b1, W2, b2):
    return pl.pallas_call(
        _mlp_body,
        out_shape=jax.ShapeDtypeStruct((B, D), jnp.float32),
    )(pooled, W1, b1.reshape(1, D), W2, b2.reshape(1, D))


def kernel(x, table, W1, b1, W2, b2):
    x_t = jnp.swapaxes(x, 0, 1).astype(jnp.int32)
    x_perm = x_t.reshape(H, NW, BPW).transpose(1, 0, 2).reshape(NW, H * BPW)
    pooled = _pool(x_perm, table)
    out = _mlp(pooled, W1, b1, W2, b2)
    return out[None, :, :]
